# baseline (speedup 1.0000x reference)
import jax
import jax.numpy as jnp
from jax.experimental import pallas as pl

C = 384
HMAP = 128
WMAP = 128
KTOP = 2048
MAXF = 4096
NUP = 4


def _maxpool3(x):
    return jax.lax.reduce_window(x, -jnp.inf, jax.lax.max, (3, 3), (1, 1), 'SAME')


def _localization(s):
    sp = jnp.pad(s, 1, mode='edge')
    di = 0.5 * (sp[2:, 1:-1] - sp[:-2, 1:-1])
    dj = 0.5 * (sp[1:-1, 2:] - sp[1:-1, :-2])
    dii = sp[2:, 1:-1] - 2.0 * s + sp[:-2, 1:-1]
    djj = sp[1:-1, 2:] - 2.0 * s + sp[1:-1, :-2]
    dij = 0.25 * (sp[2:, 2:] - sp[2:, :-2] - sp[:-2, 2:] + sp[:-2, :-2])
    det = dii * djj - dij * dij
    safe = jnp.abs(det) > 1e-6
    dets = jnp.where(safe, det, 1.0)
    disp_i = jnp.where(safe, -(djj * di - dij * dj) / dets, 2.0)
    disp_j = jnp.where(safe, -(dii * dj - dij * di) / dets, 2.0)
    return jnp.stack([disp_i, disp_j], axis=0)


def _interpolate(dense, pos):
    Hm, Wm = dense.shape[1], dense.shape[2]
    i, j = pos[0], pos[1]
    i0 = jnp.clip(jnp.floor(i), 0.0, Hm - 2.0)
    j0 = jnp.clip(jnp.floor(j), 0.0, Wm - 2.0)
    wi = i - i0
    wj = j - j0
    i0i = i0.astype(jnp.int32)
    j0i = j0.astype(jnp.int32)
    d00 = dense[:, i0i, j0i]
    d01 = dense[:, i0i, j0i + 1]
    d10 = dense[:, i0i + 1, j0i]
    d11 = dense[:, i0i + 1, j0i + 1]
    desc = (1.0 - wi) * (1.0 - wj) * d00 + (1.0 - wi) * wj * d01 + wi * (1.0 - wj) * d10 + wi * wj * d11
    valid = (i >= 0) & (i <= Hm - 1) & (j >= 0) & (j <= Wm - 1)
    return desc, valid


def _level(x_ds, W, v):
    feat = jax.nn.relu(jnp.einsum('chw,cd->dhw', x_ds, W))
    score = jax.nn.softplus(jnp.einsum('dhw,d->hw', feat, v))
    lmax = _maxpool3(score)
    s_nms = jnp.where(lmax == score, score, 0.0)
    vals, idx = jax.lax.top_k(s_nms.reshape(-1), KTOP)
    hi = idx // WMAP
    wi = idx % WMAP
    disp = _localization(score)
    di = disp[0, hi, wi]
    dj = disp[1, hi, wi]
    valid_d = (jnp.abs(di) < 0.5) & (jnp.abs(dj) < 0.5)
    kp = jnp.stack([hi.astype(jnp.float32) + di, wi.astype(jnp.float32) + dj], axis=0)
    desc, valid_b = _interpolate(feat, kp)
    valid = valid_d & valid_b & (vals > 0)
    scores_k = vals * valid
    desc = desc * valid
    desc = desc / (jnp.linalg.norm(desc, axis=0, keepdims=True) + 1e-8)
    kp_up = kp
    for _ in range(NUP):
        kp_up = kp_up * 2.0 + 0.5
    return kp_up.T, desc.T, scores_k


def _flip_body(kp_ref, out_ref):
    kp = kp_ref[:, :]
    out_ref[:, :] = jnp.concatenate([kp[:, 1:2], kp[:, 0:1]], axis=1)


def kernel(images, W_early, W_middle, W_deep, v_early, v_middle, v_deep):
    img = images[0]
    x_ds = img.reshape(3, HMAP, 4, WMAP, 4).mean(axis=(2, 4))
    kps, descs, scs = [], [], []
    for W, v in ((W_early, v_early), (W_middle, v_middle), (W_deep, v_deep)):
        k, d, s = _level(x_ds, W, v)
        kps.append(k)
        descs.append(d)
        scs.append(s)
    keypoints = jnp.concatenate(kps, axis=0)
    descriptors = jnp.concatenate(descs, axis=0)
    scores = jnp.concatenate(scs, axis=0)
    order = jnp.argsort(-scores)
    scores = scores[order][:MAXF]
    descriptors = descriptors[order][:MAXF]
    keypoints = keypoints[order][:MAXF]
    keypoints = pl.pallas_call(
        _flip_body,
        out_shape=jax.ShapeDtypeStruct((MAXF, 2), jnp.float32),
    )(keypoints)
    return keypoints, descriptors, scores


# speedup vs baseline: 1.0457x; 1.0457x over previous
import functools

import jax
import jax.numpy as jnp
from jax.experimental import pallas as pl

C = 384
HMAP = 128
WMAP = 128
HW = HMAP * WMAP
KTOP = 2048
MAXF = 4096
NLVL = 3
NEG_INF = float("-inf")


def _k0_body(x0_ref, x1_ref, x2_ref, x3_ref, out_ref):
    def sum_i(r):
        return ((r[:, :, 0, :] + r[:, :, 1, :]) + r[:, :, 2, :]) + r[:, :, 3, :]

    s0 = sum_i(x0_ref[...])
    s1 = sum_i(x1_ref[...])
    s2 = sum_i(x2_ref[...])
    s3 = sum_i(x3_ref[...])
    t0 = s0 + s2
    t1 = s1 + s3
    out_ref[...] = (t0 + t1) * (1.0 / 16.0)


def _downsample(img):
    views = [img[:, :, j::4].reshape(3, HMAP, 4, WMAP) for j in range(4)]
    return pl.pallas_call(
        _k0_body,
        out_shape=jax.ShapeDtypeStruct((3, HMAP, WMAP), jnp.float32),
    )(*views)


def _k1a_body(x_ref, w_ref, v_ref, score_ref):
    X = x_ref[...]
    Wm = w_ref[0]
    vv = v_ref[0]
    Xb = X.astype(jnp.bfloat16)
    Wb = Wm.astype(jnp.bfloat16)
    F = jax.lax.dot_general(Wb, Xb, (((0,), (0,)), ((), ())),
                            preferred_element_type=jnp.float32)
    F = jnp.maximum(F, 0.0)
    Fb = F.astype(jnp.bfloat16)
    vb = vv.astype(jnp.bfloat16)
    lg = jax.lax.dot_general(vb, Fb, (((1,), (0,)), ((), ())),
                             preferred_element_type=jnp.float32)
    score_ref[0] = jax.nn.softplus(lg)


def _score_maps(X, Wcat, vcat):
    return pl.pallas_call(
        _k1a_body,
        grid=(NLVL,),
        in_specs=[
            pl.BlockSpec((3, HW), lambda l: (0, 0)),
            pl.BlockSpec((1, 3, C), lambda l: (l, 0, 0)),
            pl.BlockSpec((1, 1, C), lambda l: (l, 0, 0)),
        ],
        out_specs=pl.BlockSpec((1, 1, HW), lambda l: (l, 0, 0)),
        out_shape=jax.ShapeDtypeStruct((NLVL, 1, HW), jnp.float32),
    )(X, Wcat, vcat)


def _k1b_body(score_ref, nms_ref, di_ref, dj_ref):
    s = score_ref[0]
    negr = jnp.full((1, WMAP), NEG_INF, jnp.float32)
    negc = jnp.full((HMAP, 1), NEG_INF, jnp.float32)
    s_dn = jnp.concatenate([s[1:, :], negr], axis=0)
    s_up = jnp.concatenate([negr, s[:-1, :]], axis=0)
    mv = jnp.maximum(jnp.maximum(s, s_dn), s_up)
    m_r = jnp.concatenate([mv[:, 1:], negc], axis=1)
    m_l = jnp.concatenate([negc, mv[:, :-1]], axis=1)
    lmax = jnp.maximum(jnp.maximum(mv, m_r), m_l)
    nms_ref[0] = jnp.where(lmax == s, s, 0.0)

    spr = jnp.concatenate([s[0:1, :], s, s[HMAP - 1:HMAP, :]], axis=0)
    sp = jnp.concatenate([spr[:, 0:1], spr, spr[:, WMAP - 1:WMAP]], axis=1)
    di = 0.5 * (sp[2:, 1:-1] - sp[:-2, 1:-1])
    dj = 0.5 * (sp[1:-1, 2:] - sp[1:-1, :-2])
    dii = sp[2:, 1:-1] - 2.0 * s + sp[:-2, 1:-1]
    djj = sp[1:-1, 2:] - 2.0 * s + sp[1:-1, :-2]
    dij = 0.25 * (sp[2:, 2:] - sp[2:, :-2] - sp[:-2, 2:] + sp[:-2, :-2])
    det = dii * djj - dij * dij
    safe = jnp.abs(det) > 1e-6
    dets = jnp.where(safe, det, 1.0)
    di_ref[0] = jnp.where(safe, -(djj * di - dij * dj) / dets, 2.0)
    dj_ref[0] = jnp.where(safe, -(dii * dj - dij * di) / dets, 2.0)


def _nms_disp(score3):
    outs = (jax.ShapeDtypeStruct((NLVL, HMAP, WMAP), jnp.float32),) * 3
    return pl.pallas_call(
        _k1b_body,
        grid=(NLVL,),
        in_specs=[pl.BlockSpec((1, HMAP, WMAP), lambda l: (l, 0, 0))],
        out_specs=(pl.BlockSpec((1, HMAP, WMAP), lambda l: (l, 0, 0)),) * 3,
        out_shape=outs,
    )(score3)


def _k3_body(xc_ref, wts_ref, lvl_ref, msk_ref, w_ref, out_ref):
    acc = jnp.zeros((out_ref.shape[1], C), jnp.float32)
    for l in range(NLVL):
        Wb = w_ref[l].astype(jnp.bfloat16)
        lacc = jnp.zeros((out_ref.shape[1], C), jnp.float32)
        for c in range(4):
            A = xc_ref[c]
            Fb = jax.lax.dot_general(A.astype(jnp.bfloat16), Wb,
                                     (((1,), (0,)), ((), ())),
                                     preferred_element_type=jnp.float32)
            Fb = jnp.maximum(Fb, 0.0)
            lacc = lacc + wts_ref[c][:, None] * Fb
        acc = acc + lvl_ref[l][:, None] * lacc
    desc = acc * msk_ref[0][:, None]
    nrm = jnp.sqrt(jnp.sum(desc * desc, axis=1, keepdims=True))
    out_ref[0] = desc / (nrm + 1e-8)


def _descriptors(xc, wts, lvl1h, vmask, Wcat):
    R = 512
    return pl.pallas_call(
        _k3_body,
        grid=(MAXF // R,),
        in_specs=[
            pl.BlockSpec((4, R, 3), lambda b: (0, b, 0)),
            pl.BlockSpec((4, R), lambda b: (0, b)),
            pl.BlockSpec((NLVL, R), lambda b: (0, b)),
            pl.BlockSpec((1, R), lambda b: (0, b)),
            pl.BlockSpec((NLVL, 3, C), lambda b: (0, 0, 0)),
        ],
        out_specs=pl.BlockSpec((1, R, C), lambda b: (b, 0, 0)),
        out_shape=jax.ShapeDtypeStruct((MAXF // R, R, C), jnp.float32),
    )(xc, wts, lvl1h, vmask, Wcat)


def kernel(images, W_early, W_middle, W_deep, v_early, v_middle, v_deep):
    img = images[0]
    x_ds = _downsample(img)
    X = x_ds.reshape(3, HW)
    Wcat = jnp.stack([W_early, W_middle, W_deep], axis=0)
    vcat = jnp.stack([v_early, v_middle, v_deep], axis=0).reshape(NLVL, 1, C)

    score3 = _score_maps(X, Wcat, vcat).reshape(NLVL, HMAP, WMAP)
    s_nms, disp_i, disp_j = _nms_disp(score3)

    vals, idx = jax.lax.top_k(s_nms.reshape(NLVL, HW), KTOP)
    hi = idx // WMAP
    wi = idx % WMAP
    di = jnp.take_along_axis(disp_i.reshape(NLVL, HW), idx, axis=1)
    dj = jnp.take_along_axis(disp_j.reshape(NLVL, HW), idx, axis=1)
    valid_d = (jnp.abs(di) < 0.5) & (jnp.abs(dj) < 0.5)
    kp_i = hi.astype(jnp.float32) + di
    kp_j = wi.astype(jnp.float32) + dj
    valid_b = (kp_i >= 0) & (kp_i <= HMAP - 1) & (kp_j >= 0) & (kp_j <= WMAP - 1)
    valid = valid_d & valid_b & (vals > 0)
    scores_c = (vals * valid).reshape(-1)

    order = jnp.argsort(-scores_c)[:MAXF]
    scores_out = scores_c[order]
    kp_i_f = kp_i.reshape(-1)[order]
    kp_j_f = kp_j.reshape(-1)[order]
    keypoints = jnp.stack([kp_j_f * 16.0 + 7.5, kp_i_f * 16.0 + 7.5], axis=1)

    lvl = order // KTOP
    i0 = jnp.clip(jnp.floor(kp_i.reshape(-1)[order]), 0.0, HMAP - 2.0)
    j0 = jnp.clip(jnp.floor(kp_j.reshape(-1)[order]), 0.0, WMAP - 2.0)
    wif = kp_i_f - i0
    wjf = kp_j_f - j0
    i0i = i0.astype(jnp.int32)
    j0i = j0.astype(jnp.int32)
    base = i0i * WMAP + j0i
    Xt = X.T
    xc = jnp.stack([Xt[base], Xt[base + 1], Xt[base + WMAP], Xt[base + WMAP + 1]],
                   axis=0)
    wts = jnp.stack([(1.0 - wif) * (1.0 - wjf), (1.0 - wif) * wjf,
                     wif * (1.0 - wjf), wif * wjf], axis=0)
    lvl1h = (lvl[None, :] == jnp.arange(NLVL)[:, None]).astype(jnp.float32)
    vmask = valid.reshape(-1)[order].astype(jnp.float32)[None, :]

    descriptors = _descriptors(xc, wts, lvl1h, vmask, Wcat).reshape(MAXF, C)
    return keypoints, descriptors, scores_out
